# Initial kernel scaffold; baseline (speedup 1.0000x reference)
import jax
import jax.numpy as jnp
from jax.experimental import pallas as pl


def kernel(x, edge_index, Wl_in, Wr_in, b_in, Wl_med, Wr_med, b_med, Wl_out, Wr_out, b_out):
    raise NotImplementedError("write your pallas kernel here")


# speedup vs baseline: 1.7965x; 1.7965x over previous
import functools

import jax
import jax.numpy as jnp
from jax import lax
from jax.experimental import pallas as pl
from jax.experimental.pallas import tpu as pltpu
from jax.experimental.pallas import tpu_sc as plsc

N_CORES = 2
N_SUB = 16
CHUNK = 128


def _sc_aggregate(table, srcm, dstm, init, half, dh, n_pass):
    cc = srcm.shape[1]
    cps = cc // N_SUB
    acc_rows = half + N_SUB
    rows_a = ((half // N_SUB + 7) // 8) * 8
    rows_b = half - (N_SUB - 1) * rows_a
    assert rows_b > 0 and rows_b % 8 == 0 and cps % 8 == 0

    mesh = plsc.VectorSubcoreMesh(core_axis_name="c", subcore_axis_name="s")

    @functools.partial(
        pl.kernel,
        out_type=jax.ShapeDtypeStruct(init.shape, jnp.float32),
        mesh=mesh,
        scratch_types=[
            pltpu.VMEM((cps, CHUNK), jnp.int32),
            pltpu.VMEM((cps, CHUNK), jnp.int32),
            pltpu.VMEM((CHUNK, dh), jnp.float32),
            pltpu.VMEM_SHARED((acc_rows, dh), jnp.float32),
            pltpu.SemaphoreType.DMA,
        ],
    )
    def k(table_hbm, src_hbm, dst_hbm, init_hbm, out_hbm,
          src_v, dst_v, rows_v, acc, sem):
        c = lax.axis_index("c")
        s = lax.axis_index("s")
        pltpu.sync_copy(src_hbm.at[c, pl.ds(s * cps, cps)], src_v)

        for r in range(n_pass):
            base = (c * n_pass + r) * half
            pltpu.sync_copy(dst_hbm.at[c, r, pl.ds(s * cps, cps)], dst_v)

            @pl.when(s < N_SUB - 1)
            def _():
                pltpu.sync_copy(init_hbm.at[pl.ds(base + s * rows_a, rows_a)],
                                acc.at[pl.ds(s * rows_a, rows_a)])

            @pl.when(s == N_SUB - 1)
            def _():
                tb = (N_SUB - 1) * rows_a
                pltpu.sync_copy(init_hbm.at[pl.ds(base + tb, rows_b)],
                                acc.at[pl.ds(tb, rows_b)])

            plsc.subcore_barrier()

            def body(j, carry):
                pltpu.async_copy(table_hbm.at[src_v.at[j]], rows_v, sem).wait()
                pltpu.sync_copy(rows_v, acc.at[dst_v.at[j]], add=True)
                return carry

            lax.fori_loop(0, cps, body, 0)

            plsc.subcore_barrier()

            @pl.when(s < N_SUB - 1)
            def _():
                pltpu.sync_copy(acc.at[pl.ds(s * rows_a, rows_a)],
                                out_hbm.at[pl.ds(base + s * rows_a, rows_a)])

            @pl.when(s == N_SUB - 1)
            def _():
                tb = (N_SUB - 1) * rows_a
                pltpu.sync_copy(acc.at[pl.ds(tb, rows_b)],
                                out_hbm.at[pl.ds(base + tb, rows_b)])

    return k(table, srcm, dstm, init)


def _split_w(W, oh):
    return W.reshape(W.shape[0], 2, oh).transpose(1, 0, 2)


def _tc_sage(aggs, hs, Wl2, Wr2, b2, act, bn=1000):
    pa, n, wa = aggs.shape
    ph, _, wh = hs.shape
    oh = Wl2.shape[2]
    hi = lax.Precision.HIGHEST

    def body(agg_ref, h_ref, wl_ref, wr_ref, b_ref, o_ref):
        acc = b_ref[0]
        for p in range(pa):
            acc = acc + jnp.dot(agg_ref[p], wl_ref[0, p * wa:(p + 1) * wa, :],
                                preferred_element_type=jnp.float32,
                                precision=hi)
        for p in range(ph):
            acc = acc + jnp.dot(h_ref[p], wr_ref[0, p * wh:(p + 1) * wh, :],
                                preferred_element_type=jnp.float32,
                                precision=hi)
        o_ref[0] = jnp.tanh(acc) if act else acc

    return pl.pallas_call(
        body,
        grid=(n // bn, 2),
        in_specs=[
            pl.BlockSpec((pa, bn, wa), lambda i, j: (0, i, 0)),
            pl.BlockSpec((ph, bn, wh), lambda i, j: (0, i, 0)),
            pl.BlockSpec((1, pa * wa, oh), lambda i, j: (j, 0, 0)),
            pl.BlockSpec((1, ph * wh, oh), lambda i, j: (j, 0, 0)),
            pl.BlockSpec((1, 1, oh), lambda i, j: (j, 0, 0)),
        ],
        out_specs=pl.BlockSpec((1, bn, oh), lambda i, j: (j, i, 0)),
        out_shape=jax.ShapeDtypeStruct((2, n, oh), jnp.float32),
    )(aggs, hs, Wl2, Wr2, b2)


def _tc_proj(hs, Wl, Wr, b1, bn=1000):
    ph, n, wh = hs.shape
    dout = Wl.shape[1]
    hi = lax.Precision.HIGHEST

    def body(h_ref, wl_ref, wr_ref, b_ref, p_ref, q_ref):
        p = jnp.zeros((bn, dout), jnp.float32)
        q = b_ref[...]
        for k in range(ph):
            p = p + jnp.dot(h_ref[k], wl_ref[k * wh:(k + 1) * wh, :],
                            preferred_element_type=jnp.float32, precision=hi)
            q = q + jnp.dot(h_ref[k], wr_ref[k * wh:(k + 1) * wh, :],
                            preferred_element_type=jnp.float32, precision=hi)
        p_ref[...] = p
        q_ref[...] = q

    return pl.pallas_call(
        body,
        grid=(n // bn,),
        in_specs=[
            pl.BlockSpec((ph, bn, wh), lambda i: (0, i, 0)),
            pl.BlockSpec((ph * wh, dout), lambda i: (0, 0)),
            pl.BlockSpec((ph * wh, dout), lambda i: (0, 0)),
            pl.BlockSpec((1, dout), lambda i: (0, 0)),
        ],
        out_specs=[
            pl.BlockSpec((bn, dout), lambda i: (i, 0)),
            pl.BlockSpec((bn, dout), lambda i: (i, 0)),
        ],
        out_shape=[
            jax.ShapeDtypeStruct((n, dout), jnp.float32),
            jax.ShapeDtypeStruct((n, dout), jnp.float32),
        ],
    )(hs, Wl, Wr, b1)


def kernel(x, edge_index, Wl_in, Wr_in, b_in, Wl_med, Wr_med, b_med,
           Wl_out, Wr_out, b_out):
    n, d_in = x.shape
    e = edge_index.shape[1]
    d_hid = Wl_in.shape[1]
    d_out = Wl_out.shape[1]
    oh = d_hid // 2
    half = n // 2

    src = edge_index[0]
    dst = edge_index[1]
    per = N_SUB * CHUNK * 8
    e_pad = ((e + per - 1) // per) * per
    pad = e_pad - e
    nch = e_pad // CHUNK
    cps = nch // N_SUB
    src_p = jnp.concatenate([src, jnp.zeros((pad,), jnp.int32)])
    dst_p = jnp.concatenate([dst, jnp.full((pad,), n, jnp.int32)])

    sub_id = (jnp.arange(e_pad, dtype=jnp.int32) // CHUNK) // cps
    junk = half + sub_id

    def dst_for(r):
        lo = r * half
        in_r = (dst_p >= lo) & (dst_p < lo + half)
        return jnp.where(in_r, dst_p - lo, junk)

    d0 = dst_for(0).reshape(nch, CHUNK)
    d1 = dst_for(1).reshape(nch, CHUNK)
    src_m1 = jnp.stack([src_p, src_p]).reshape(2, nch, CHUNK)
    dst_m1 = jnp.stack([d0, d1]).reshape(2, 1, nch, CHUNK)
    src_m2 = jnp.stack([src_p, src_p + n]).reshape(2, nch, CHUNK)
    dst_m2 = jnp.stack([jnp.stack([d0, d1]), jnp.stack([d0, d1])])

    agg = _sc_aggregate(x, src_m1, dst_m1, jnp.zeros((n, d_in), jnp.float32),
                        half, d_in, 1)
    h = _tc_sage(agg[None], x[None], _split_w(Wl_in, oh), _split_w(Wr_in, oh),
                 b_in.reshape(2, 1, -1), act=True)

    Wl_med2 = _split_w(Wl_med, oh)
    Wr_med2 = _split_w(Wr_med, oh)
    b_med2 = b_med.reshape(2, 1, -1)
    z_hid = jnp.zeros((2 * n, oh), jnp.float32)
    for _ in range(2):
        agg = _sc_aggregate(h.reshape(2 * n, oh), src_m2, dst_m2, z_hid,
                            half, oh, 2)
        h = _tc_sage(agg.reshape(2, n, oh), h, Wl_med2, Wr_med2, b_med2,
                     act=True)

    p, q = _tc_proj(h, Wl_out, Wr_out, b_out.reshape(1, -1))
    return _sc_aggregate(p, src_m1, dst_m1, q, half, d_out, 1)


# speedup vs baseline: 2.1037x; 1.1710x over previous
import functools

import jax
import jax.numpy as jnp
from jax import lax
from jax.experimental import pallas as pl
from jax.experimental.pallas import tpu as pltpu
from jax.experimental.pallas import tpu_sc as plsc

N_CORES = 2
N_SUB = 16
CHUNK = 128


def _sc_aggregate(table, srcm, dstm, init, half, dh, n_pass):
    cc = srcm.shape[1]
    cps = cc // N_SUB
    assert cps % 2 == 0
    acc_rows = half + N_SUB
    rows_a = ((half // N_SUB + 7) // 8) * 8
    rows_b = half - (N_SUB - 1) * rows_a
    assert rows_b > 0 and rows_b % 8 == 0 and cps % 8 == 0

    mesh = plsc.VectorSubcoreMesh(core_axis_name="c", subcore_axis_name="s")

    @functools.partial(
        pl.kernel,
        out_type=jax.ShapeDtypeStruct(init.shape, jnp.float32),
        mesh=mesh,
        scratch_types=[
            pltpu.VMEM((cps, CHUNK), jnp.int32),
            pltpu.VMEM((cps, CHUNK), jnp.int32),
            pltpu.VMEM((CHUNK, dh), jnp.float32),
            pltpu.VMEM((CHUNK, dh), jnp.float32),
            pltpu.VMEM_SHARED((acc_rows, dh), jnp.float32),
            pltpu.SemaphoreType.DMA,
            pltpu.SemaphoreType.DMA,
        ],
    )
    def k(table_hbm, src_hbm, dst_hbm, init_hbm, out_hbm,
          src_v, dst_v, rows0_v, rows1_v, acc, sem0, sem1):
        c = lax.axis_index("c")
        s = lax.axis_index("s")
        pltpu.sync_copy(src_hbm.at[c, pl.ds(s * cps, cps)], src_v)

        for r in range(n_pass):
            base = (c * n_pass + r) * half
            pltpu.sync_copy(dst_hbm.at[c, r, pl.ds(s * cps, cps)], dst_v)

            @pl.when(s < N_SUB - 1)
            def _():
                pltpu.sync_copy(init_hbm.at[pl.ds(base + s * rows_a, rows_a)],
                                acc.at[pl.ds(s * rows_a, rows_a)])

            @pl.when(s == N_SUB - 1)
            def _():
                tb = (N_SUB - 1) * rows_a
                pltpu.sync_copy(init_hbm.at[pl.ds(base + tb, rows_b)],
                                acc.at[pl.ds(tb, rows_b)])

            plsc.subcore_barrier()

            pltpu.async_copy(table_hbm.at[src_v.at[0]], rows0_v, sem0)

            def body(jj, carry):
                j = 2 * jj
                pltpu.async_copy(table_hbm.at[src_v.at[j + 1]], rows1_v, sem1)
                pltpu.make_async_copy(table_hbm.at[src_v.at[j]], rows0_v,
                                      sem0).wait()
                pltpu.sync_copy(rows0_v, acc.at[dst_v.at[j]], add=True)
                j2 = jnp.minimum(j + 2, cps - 1)
                pltpu.async_copy(table_hbm.at[src_v.at[j2]], rows0_v, sem0)
                pltpu.make_async_copy(table_hbm.at[src_v.at[j + 1]], rows1_v,
                                      sem1).wait()
                pltpu.sync_copy(rows1_v, acc.at[dst_v.at[j + 1]], add=True)
                return carry

            lax.fori_loop(0, cps // 2, body, 0)
            pltpu.make_async_copy(table_hbm.at[src_v.at[0]], rows0_v,
                                  sem0).wait()

            plsc.subcore_barrier()

            @pl.when(s < N_SUB - 1)
            def _():
                pltpu.sync_copy(acc.at[pl.ds(s * rows_a, rows_a)],
                                out_hbm.at[pl.ds(base + s * rows_a, rows_a)])

            @pl.when(s == N_SUB - 1)
            def _():
                tb = (N_SUB - 1) * rows_a
                pltpu.sync_copy(acc.at[pl.ds(tb, rows_b)],
                                out_hbm.at[pl.ds(base + tb, rows_b)])

    return k(table, srcm, dstm, init)


def _split_w(W, oh):
    return W.reshape(W.shape[0], 2, oh).transpose(1, 0, 2)


def _tc_sage(aggs, hs, Wl2, Wr2, b2, act, bn=1000):
    pa, n, wa = aggs.shape
    ph, _, wh = hs.shape
    oh = Wl2.shape[2]
    hi = lax.Precision.HIGHEST

    def body(agg_ref, h_ref, wl_ref, wr_ref, b_ref, o_ref):
        acc = b_ref[0]
        for p in range(pa):
            acc = acc + jnp.dot(agg_ref[p], wl_ref[0, p * wa:(p + 1) * wa, :],
                                preferred_element_type=jnp.float32,
                                precision=hi)
        for p in range(ph):
            acc = acc + jnp.dot(h_ref[p], wr_ref[0, p * wh:(p + 1) * wh, :],
                                preferred_element_type=jnp.float32,
                                precision=hi)
        o_ref[0] = jnp.tanh(acc) if act else acc

    return pl.pallas_call(
        body,
        grid=(n // bn, 2),
        in_specs=[
            pl.BlockSpec((pa, bn, wa), lambda i, j: (0, i, 0)),
            pl.BlockSpec((ph, bn, wh), lambda i, j: (0, i, 0)),
            pl.BlockSpec((1, pa * wa, oh), lambda i, j: (j, 0, 0)),
            pl.BlockSpec((1, ph * wh, oh), lambda i, j: (j, 0, 0)),
            pl.BlockSpec((1, 1, oh), lambda i, j: (j, 0, 0)),
        ],
        out_specs=pl.BlockSpec((1, bn, oh), lambda i, j: (j, i, 0)),
        out_shape=jax.ShapeDtypeStruct((2, n, oh), jnp.float32),
    )(aggs, hs, Wl2, Wr2, b2)


def _tc_proj(hs, Wl, Wr, b1, bn=1000):
    ph, n, wh = hs.shape
    dout = Wl.shape[1]
    hi = lax.Precision.HIGHEST

    def body(h_ref, wl_ref, wr_ref, b_ref, p_ref, q_ref):
        p = jnp.zeros((bn, dout), jnp.float32)
        q = b_ref[...]
        for k in range(ph):
            p = p + jnp.dot(h_ref[k], wl_ref[k * wh:(k + 1) * wh, :],
                            preferred_element_type=jnp.float32, precision=hi)
            q = q + jnp.dot(h_ref[k], wr_ref[k * wh:(k + 1) * wh, :],
                            preferred_element_type=jnp.float32, precision=hi)
        p_ref[...] = p
        q_ref[...] = q

    return pl.pallas_call(
        body,
        grid=(n // bn,),
        in_specs=[
            pl.BlockSpec((ph, bn, wh), lambda i: (0, i, 0)),
            pl.BlockSpec((ph * wh, dout), lambda i: (0, 0)),
            pl.BlockSpec((ph * wh, dout), lambda i: (0, 0)),
            pl.BlockSpec((1, dout), lambda i: (0, 0)),
        ],
        out_specs=[
            pl.BlockSpec((bn, dout), lambda i: (i, 0)),
            pl.BlockSpec((bn, dout), lambda i: (i, 0)),
        ],
        out_shape=[
            jax.ShapeDtypeStruct((n, dout), jnp.float32),
            jax.ShapeDtypeStruct((n, dout), jnp.float32),
        ],
    )(hs, Wl, Wr, b1)


def kernel(x, edge_index, Wl_in, Wr_in, b_in, Wl_med, Wr_med, b_med,
           Wl_out, Wr_out, b_out):
    n, d_in = x.shape
    e = edge_index.shape[1]
    d_hid = Wl_in.shape[1]
    d_out = Wl_out.shape[1]
    oh = d_hid // 2
    half = n // 2

    src = edge_index[0]
    dst = edge_index[1]
    per = N_SUB * CHUNK * 8
    e_pad = ((e + per - 1) // per) * per
    pad = e_pad - e
    nch = e_pad // CHUNK
    cps = nch // N_SUB
    src_p = jnp.concatenate([src, jnp.zeros((pad,), jnp.int32)])
    dst_p = jnp.concatenate([dst, jnp.full((pad,), n, jnp.int32)])

    sub_id = (jnp.arange(e_pad, dtype=jnp.int32) // CHUNK) // cps
    junk = half + sub_id

    def dst_for(r):
        lo = r * half
        in_r = (dst_p >= lo) & (dst_p < lo + half)
        return jnp.where(in_r, dst_p - lo, junk)

    d0 = dst_for(0).reshape(nch, CHUNK)
    d1 = dst_for(1).reshape(nch, CHUNK)
    src_m1 = jnp.stack([src_p, src_p]).reshape(2, nch, CHUNK)
    dst_m1 = jnp.stack([d0, d1]).reshape(2, 1, nch, CHUNK)
    src_m2 = jnp.stack([src_p, src_p + n]).reshape(2, nch, CHUNK)
    dst_m2 = jnp.stack([jnp.stack([d0, d1]), jnp.stack([d0, d1])])

    agg = _sc_aggregate(x, src_m1, dst_m1, jnp.zeros((n, d_in), jnp.float32),
                        half, d_in, 1)
    h = _tc_sage(agg[None], x[None], _split_w(Wl_in, oh), _split_w(Wr_in, oh),
                 b_in.reshape(2, 1, -1), act=True)

    Wl_med2 = _split_w(Wl_med, oh)
    Wr_med2 = _split_w(Wr_med, oh)
    b_med2 = b_med.reshape(2, 1, -1)
    z_hid = jnp.zeros((2 * n, oh), jnp.float32)
    for _ in range(2):
        agg = _sc_aggregate(h.reshape(2 * n, oh), src_m2, dst_m2, z_hid,
                            half, oh, 2)
        h = _tc_sage(agg.reshape(2, n, oh), h, Wl_med2, Wr_med2, b_med2,
                     act=True)

    p, q = _tc_proj(h, Wl_out, Wr_out, b_out.reshape(1, -1))
    return _sc_aggregate(p, src_m1, dst_m1, q, half, d_out, 1)
